# Initial kernel scaffold; baseline (speedup 1.0000x reference)
import jax
import jax.numpy as jnp
from jax.experimental import pallas as pl


def kernel(x, table, weights, hash0_coeffs, hash1_coeffs):
    raise NotImplementedError("write your pallas kernel here")


# speedup vs baseline: 3.2960x; 3.2960x over previous
import functools

import jax
import jax.numpy as jnp
from jax import lax
from jax.experimental import pallas as pl
from jax.experimental.pallas import tpu as pltpu
from jax.experimental.pallas import tpu_sc as plsc

P31 = (1 << 31) - 1
N_ROWS = 125000
N_WEIGHTS = 8_000_000
BATCH = 16384
DIM = 64
NH = 4
NC, NS, L = 2, 16, 16
NW = NC * NS
PER_W = BATCH // NW
CHUNK = 128
NCHUNK = PER_W // CHUNK
NDC = DIM // L


def _fold2(v):
    m = jnp.uint32(P31)
    v = (v & m) + (v >> jnp.uint32(31))
    v = (v & m) + (v >> jnp.uint32(31))
    return v


def _mulmod_pow2(t, s):
    m = jnp.uint32(P31)
    lo = (t << jnp.uint32(s)) & m
    hi = t >> jnp.uint32(31 - s)
    return _fold2(lo + hi)


def _mod_r(acc, r_mod):
    acc_i = acc.astype(jnp.int32)
    rinv = jnp.float32(1.0) / jnp.float32(r_mod)
    q = (acc_i.astype(jnp.float32) * rinv).astype(jnp.int32)
    r = acc_i - q * jnp.int32(r_mod)
    for _ in range(2):
        r = jnp.where(r < 0, r + jnp.int32(r_mod), r)
    for _ in range(2):
        r = jnp.where(r >= jnp.int32(r_mod), r - jnp.int32(r_mod), r)
    return r


def _hash_one(x1, x0, a, b, r_mod):
    a1 = a >> jnp.uint32(16)
    a0 = a & jnp.uint32(0xFFFF)
    t1 = _mulmod_pow2(a1 * x1, 26)
    t2 = _mulmod_pow2(a1 * x0, 16)
    t3 = _mulmod_pow2(a0 * x1, 10)
    t4 = a0 * x0
    acc = _fold2(t1 + t2)
    acc = _fold2(acc + t3)
    acc = _fold2(acc + t4)
    acc = _fold2(acc + b)
    acc = jnp.where(acc >= jnp.uint32(P31), acc - jnp.uint32(P31), acc)
    return _mod_r(acc, r_mod)


def _hash_body(x_ref, c0_ref, c1_ref, i0_ref, i1r_ref, i1c_ref):
    x = x_ref[...].astype(jnp.uint32)
    x1 = x >> jnp.uint32(10)
    x0 = x & jnp.uint32(0x3FF)
    for i in range(NH):
        a0c = c0_ref[i, 0].astype(jnp.uint32)
        b0c = c0_ref[i, 1].astype(jnp.uint32)
        i0_ref[i] = _hash_one(x1, x0, a0c, b0c, N_ROWS)
        a1c = c1_ref[i, 0].astype(jnp.uint32)
        b1c = c1_ref[i, 1].astype(jnp.uint32)
        h1 = _hash_one(x1, x0, a1c, b1c, N_WEIGHTS)
        i1r_ref[i] = h1 >> 4
        i1c_ref[i] = h1 & 15


def _hash_call(xb, c0, c1, interpret=False):
    return pl.pallas_call(
        _hash_body,
        out_shape=[
            jax.ShapeDtypeStruct((NH, 128, 128), jnp.int32),
            jax.ShapeDtypeStruct((NH, 128, 128), jnp.int32),
            jax.ShapeDtypeStruct((NH, 128, 128), jnp.int32),
        ],
        in_specs=[
            pl.BlockSpec(memory_space=pltpu.VMEM),
            pl.BlockSpec(memory_space=pltpu.SMEM),
            pl.BlockSpec(memory_space=pltpu.SMEM),
        ],
        interpret=interpret,
    )(xb, c0, c1)


def _sc_body(table_hbm, w16_hbm, i0_hbm, i1r_hbm, i1c_hbm, out_hbm,
             idx0_v, i1r_v, i1c_v, rows_v, wrow_v, out_v, sem):
    wid = lax.axis_index("s") * jnp.int32(NC) + lax.axis_index("c")
    base = wid * jnp.int32(PER_W)

    @pl.loop(jnp.int32(0), jnp.int32(NCHUNK))
    def _chunk(c):
        off = base + c * jnp.int32(CHUNK)
        cps = []
        for i in range(NH):
            i32 = jnp.int32(i)
            cps.append(pltpu.async_copy(
                i0_hbm.at[i32, pl.ds(off, CHUNK)], idx0_v.at[i32], sem))
            cps.append(pltpu.async_copy(
                i1r_hbm.at[i32, pl.ds(off, CHUNK)], i1r_v.at[i32], sem))
            cps.append(pltpu.async_copy(
                i1c_hbm.at[i32, pl.ds(off, CHUNK)], i1c_v.at[i32], sem))
        for cp_ in cps:
            cp_.wait()

        gs = []
        for i in range(NH):
            i32 = jnp.int32(i)
            gs.append(pltpu.async_copy(
                table_hbm.at[idx0_v.at[i32]], rows_v.at[i32], sem))
            gs.append(pltpu.async_copy(
                w16_hbm.at[i1r_v.at[i32]], wrow_v.at[i32], sem))
        for cp_ in gs:
            cp_.wait()

        iota16 = lax.iota(jnp.int32, L)

        @pl.loop(jnp.int32(0), jnp.int32(CHUNK // L))
        def _grp(jb):
            jbase = jb * jnp.int32(L)
            jvec = jbase + iota16
            wvs = []
            for i in range(NH):
                i32 = jnp.int32(i)
                colv = i1c_v[i32, pl.ds(jbase, L)]
                wvs.append(plsc.load_gather(
                    wrow_v, [jnp.full((L,), i, jnp.int32), jvec, colv]))
            for k in range(L):
                j = jbase + jnp.int32(k)
                accs = [None] * NDC
                for i in range(NH):
                    wb = jnp.full((L,), wvs[i][k], jnp.float32)
                    for dc in range(NDC):
                        r = rows_v[jnp.int32(i), j, pl.ds(dc * L, L)]
                        contrib = wb * r
                        accs[dc] = contrib if accs[dc] is None else accs[dc] + contrib
                for dc in range(NDC):
                    out_v[j, pl.ds(dc * L, L)] = accs[dc]

        pltpu.sync_copy(out_v, out_hbm.at[pl.ds(off, CHUNK), :])


def _sc_call(table, w16, i0, i1r, i1c):
    mesh = plsc.VectorSubcoreMesh(core_axis_name="c", subcore_axis_name="s",
                                  num_cores=NC, num_subcores=NS)
    cp = pltpu.CompilerParams(needs_layout_passes=False,
                              use_tc_tiling_on_sc=False)
    f = pl.kernel(
        _sc_body,
        out_type=jax.ShapeDtypeStruct((BATCH, DIM), jnp.float32),
        mesh=mesh,
        scratch_types=[
            pltpu.VMEM((NH, CHUNK), jnp.int32),
            pltpu.VMEM((NH, CHUNK), jnp.int32),
            pltpu.VMEM((NH, CHUNK), jnp.int32),
            pltpu.VMEM((NH, CHUNK, DIM), jnp.float32),
            pltpu.VMEM((NH, CHUNK, L), jnp.float32),
            pltpu.VMEM((CHUNK, DIM), jnp.float32),
            pltpu.SemaphoreType.DMA,
        ],
        compiler_params=cp,
    )
    return f(table, w16, i0, i1r, i1c)


def kernel(x, table, weights, hash0_coeffs, hash1_coeffs):
    x32 = x.astype(jnp.int32)
    c0 = hash0_coeffs.astype(jnp.int32)
    c1 = hash1_coeffs.astype(jnp.int32)
    xb = x32.reshape(128, 128)
    i0, i1r, i1c = _hash_call(xb, c0, c1)
    i0 = i0.reshape(NH, BATCH)
    i1r = i1r.reshape(NH, BATCH)
    i1c = i1c.reshape(NH, BATCH)
    w16 = weights.reshape(N_WEIGHTS // L, L)
    return _sc_call(table, w16, i0, i1r, i1c)


# speedup vs baseline: 3.4631x; 1.0507x over previous
import functools

import jax
import jax.numpy as jnp
from jax import lax
from jax.experimental import pallas as pl
from jax.experimental.pallas import tpu as pltpu
from jax.experimental.pallas import tpu_sc as plsc

P31 = (1 << 31) - 1
N_ROWS = 125000
N_WEIGHTS = 8_000_000
BATCH = 16384
DIM = 64
NH = 4
NC, NS, L = 2, 16, 16
NW = NC * NS
PER_W = BATCH // NW
CHUNK = 128
NCHUNK = PER_W // CHUNK
NDC = DIM // L


def _fold2(v):
    m = jnp.uint32(P31)
    v = (v & m) + (v >> jnp.uint32(31))
    v = (v & m) + (v >> jnp.uint32(31))
    return v


def _mulmod_pow2(t, s):
    m = jnp.uint32(P31)
    lo = (t << jnp.uint32(s)) & m
    hi = t >> jnp.uint32(31 - s)
    return _fold2(lo + hi)


def _mod_r(acc, r_mod):
    acc_i = acc.astype(jnp.int32)
    rinv = jnp.float32(1.0) / jnp.float32(r_mod)
    q = (acc_i.astype(jnp.float32) * rinv).astype(jnp.int32)
    r = acc_i - q * jnp.int32(r_mod)
    for _ in range(2):
        r = jnp.where(r < 0, r + jnp.int32(r_mod), r)
    for _ in range(2):
        r = jnp.where(r >= jnp.int32(r_mod), r - jnp.int32(r_mod), r)
    return r


def _hash_one(x1, x0, a, b, r_mod):
    a1 = a >> jnp.uint32(16)
    a0 = a & jnp.uint32(0xFFFF)
    t1 = _mulmod_pow2(a1 * x1, 26)
    t2 = _mulmod_pow2(a1 * x0, 16)
    t3 = _mulmod_pow2(a0 * x1, 10)
    t4 = a0 * x0
    acc = _fold2(t1 + t2)
    acc = _fold2(acc + t3)
    acc = _fold2(acc + t4)
    acc = _fold2(acc + b)
    acc = jnp.where(acc >= jnp.uint32(P31), acc - jnp.uint32(P31), acc)
    return _mod_r(acc, r_mod)


def _hash_body(x_ref, c0_ref, c1_ref, idx_ref):
    x = x_ref[...].astype(jnp.uint32)
    x1 = x >> jnp.uint32(10)
    x0 = x & jnp.uint32(0x3FF)
    for i in range(NH):
        a0c = c0_ref[i, 0].astype(jnp.uint32)
        b0c = c0_ref[i, 1].astype(jnp.uint32)
        idx_ref[i] = _hash_one(x1, x0, a0c, b0c, N_ROWS)
        a1c = c1_ref[i, 0].astype(jnp.uint32)
        b1c = c1_ref[i, 1].astype(jnp.uint32)
        h1 = _hash_one(x1, x0, a1c, b1c, N_WEIGHTS)
        idx_ref[NH + i] = h1 >> 4
        idx_ref[2 * NH + i] = h1 & 15


def _hash_call(xb, c0, c1, interpret=False):
    return pl.pallas_call(
        _hash_body,
        out_shape=jax.ShapeDtypeStruct((3 * NH, 128, 128), jnp.int32),
        in_specs=[
            pl.BlockSpec(memory_space=pltpu.VMEM),
            pl.BlockSpec(memory_space=pltpu.SMEM),
            pl.BlockSpec(memory_space=pltpu.SMEM),
        ],
        interpret=interpret,
    )(xb, c0, c1)


def _issue_idx(idx_hbm, off, idx_v, sem):
    cps = []
    for i in range(3 * NH):
        i32 = jnp.int32(i)
        cps.append(pltpu.async_copy(
            idx_hbm.at[i32, pl.ds(off, CHUNK)], idx_v.at[i32], sem))
    return cps


def _issue_gathers(table_hbm, w16_hbm, idx_v, rows_v, wrow_v, sem):
    gs = []
    for i in range(NH):
        gs.append(pltpu.async_copy(
            table_hbm.at[idx_v.at[jnp.int32(i)]], rows_v.at[jnp.int32(i)], sem))
        gs.append(pltpu.async_copy(
            w16_hbm.at[idx_v.at[jnp.int32(NH + i)]], wrow_v.at[jnp.int32(i)], sem))
    return gs


def _combine(idx_v, rows_v, wrow_v, out_v):
    iota16 = lax.iota(jnp.int32, L)

    @pl.loop(jnp.int32(0), jnp.int32(CHUNK // L))
    def _grp(jb):
        jbase = jb * jnp.int32(L)
        jvec = jbase + iota16
        wvs = []
        for i in range(NH):
            colv = idx_v[jnp.int32(2 * NH + i), pl.ds(jbase, L)]
            wvs.append(plsc.load_gather(
                wrow_v, [jnp.full((L,), i, jnp.int32), jvec, colv]))
        for k in range(L):
            j = jbase + jnp.int32(k)
            accs = [None] * NDC
            for i in range(NH):
                wb = jnp.full((L,), wvs[i][k], jnp.float32)
                for dc in range(NDC):
                    r = rows_v[jnp.int32(i), j, pl.ds(dc * L, L)]
                    contrib = wb * r
                    accs[dc] = contrib if accs[dc] is None else accs[dc] + contrib
            for dc in range(NDC):
                out_v[j, pl.ds(dc * L, L)] = accs[dc]


def _sc_body(table_hbm, w16_hbm, idx_hbm, out_hbm,
             idx_v0, idx_v1, rows_v0, rows_v1, wrow_v0, wrow_v1,
             out_v0, out_v1,
             sem_i0, sem_i1, sem_g0, sem_g1, sem_o0, sem_o1):
    wid = lax.axis_index("s") * jnp.int32(NC) + lax.axis_index("c")
    base = wid * jnp.int32(PER_W)
    idx_v = (idx_v0, idx_v1)
    rows_v = (rows_v0, rows_v1)
    wrow_v = (wrow_v0, wrow_v1)
    out_v = (out_v0, out_v1)
    sem_i = (sem_i0, sem_i1)
    sem_g = (sem_g0, sem_g1)
    sem_o = (sem_o0, sem_o1)

    def off(c):
        return base + jnp.int32(c * CHUNK)

    for cp_ in _issue_idx(idx_hbm, off(0), idx_v[0], sem_i[0]):
        cp_.wait()
    gathers = _issue_gathers(table_hbm, w16_hbm, idx_v[0], rows_v[0],
                             wrow_v[0], sem_g[0])
    idx_cps = None
    if NCHUNK > 1:
        idx_cps = _issue_idx(idx_hbm, off(1), idx_v[1], sem_i[1])
    out_cp = None
    for c in range(NCHUNK):
        p = c & 1
        for cp_ in gathers:
            cp_.wait()
        if c + 1 < NCHUNK:
            for cp_ in idx_cps:
                cp_.wait()
            gathers = _issue_gathers(table_hbm, w16_hbm, idx_v[(c + 1) & 1],
                                     rows_v[(c + 1) & 1], wrow_v[(c + 1) & 1],
                                     sem_g[(c + 1) & 1])
        _combine(idx_v[p], rows_v[p], wrow_v[p], out_v[p])
        if c + 2 < NCHUNK:
            idx_cps = _issue_idx(idx_hbm, off(c + 2), idx_v[c & 1], sem_i[c & 1])
        if out_cp is not None:
            out_cp.wait()
        out_cp = pltpu.async_copy(
            out_v[p], out_hbm.at[pl.ds(off(c), CHUNK), :], sem_o[p])
    out_cp.wait()


def _sc_call(table, w16, idx):
    mesh = plsc.VectorSubcoreMesh(core_axis_name="c", subcore_axis_name="s",
                                  num_cores=NC, num_subcores=NS)
    cp = pltpu.CompilerParams(needs_layout_passes=False,
                              use_tc_tiling_on_sc=False)
    f = pl.kernel(
        _sc_body,
        out_type=jax.ShapeDtypeStruct((BATCH, DIM), jnp.float32),
        mesh=mesh,
        scratch_types=[
            pltpu.VMEM((3 * NH, CHUNK), jnp.int32),
            pltpu.VMEM((3 * NH, CHUNK), jnp.int32),
            pltpu.VMEM((NH, CHUNK, DIM), jnp.float32),
            pltpu.VMEM((NH, CHUNK, DIM), jnp.float32),
            pltpu.VMEM((NH, CHUNK, L), jnp.float32),
            pltpu.VMEM((NH, CHUNK, L), jnp.float32),
            pltpu.VMEM((CHUNK, DIM), jnp.float32),
            pltpu.VMEM((CHUNK, DIM), jnp.float32),
            pltpu.SemaphoreType.DMA,
            pltpu.SemaphoreType.DMA,
            pltpu.SemaphoreType.DMA,
            pltpu.SemaphoreType.DMA,
            pltpu.SemaphoreType.DMA,
            pltpu.SemaphoreType.DMA,
        ],
        compiler_params=cp,
    )
    return f(table, w16, idx)


def kernel(x, table, weights, hash0_coeffs, hash1_coeffs):
    x32 = x.astype(jnp.int32)
    c0 = hash0_coeffs.astype(jnp.int32)
    c1 = hash1_coeffs.astype(jnp.int32)
    xb = x32.reshape(128, 128)
    idx = _hash_call(xb, c0, c1).reshape(3 * NH, BATCH)
    w16 = weights.reshape(N_WEIGHTS // L, L)
    return _sc_call(table, w16, idx)
